# baseline (speedup 1.0000x reference)
import jax
import jax.numpy as jnp
from jax import lax
from jax.experimental import pallas as pl
from jax.experimental.pallas import tpu as pltpu
from jax.experimental.pallas import tpu_sc as plsc

NC = 2
NS = 16
NW = NC * NS
LANES = 16

B = 16384
L = 50
E = 64
EV = E // LANES
BW = B // NW
CB = 256
NBLK = CB // LANES
SUB = 128
LN_EPS = 1e-5


def _rsqrt(x):
    i = plsc.bitcast(x, jnp.int32)
    y = plsc.bitcast(jnp.int32(0x5F3759DF) - (i >> 1), jnp.float32)
    for _ in range(3):
        y = y * (1.5 - 0.5 * x * y * y)
    return y


def _body(idx_hbm, info_hbm, table_hbm, gamma_hbm, beta_hbm, out_hbm,
          idx_all, info_all, idx_buf, rows_v, gb_v, stg_v, sums_v, ab_v, sem):
    wid = lax.axis_index("s") * NC + lax.axis_index("c")
    nw_base = pl.multiple_of(wid * (BW * L), BW * L)

    pltpu.sync_copy(idx_hbm.at[pl.ds(nw_base, BW * L)], idx_all)
    pltpu.sync_copy(info_hbm.at[pl.ds(nw_base, BW * L)], info_all)
    pltpu.sync_copy(gamma_hbm, gb_v.at[0])
    pltpu.sync_copy(beta_hbm, gb_v.at[1])
    gammas = [gb_v[0, pl.ds(p * LANES, LANES)] for p in range(EV)]
    betas = [gb_v[1, pl.ds(p * LANES, LANES)] for p in range(EV)]

    iota16 = lax.iota(jnp.int32, LANES)
    iota_l = iota16 * L

    svecs = [((16 * p + iota16) // 8) * 2048 + ((16 * p + iota16) % 8) * 128
             for p in range(EV)]

    def chunk_body(c, carry):
        l = c // 2
        sb = c % 2
        cbase = sb * (CB * L) + l

        def idxb_body(k, carry2):
            pvec = iota_l + (cbase + k * (LANES * L))
            idx_buf[pl.ds(k * LANES, LANES)] = plsc.load_gather(idx_all, [pvec])
            return carry2

        lax.fori_loop(0, NBLK, idxb_body, 0)

        cps = [pltpu.async_copy(table_hbm.at[idx_buf.at[pl.ds(k * SUB, SUB)]],
                                rows_v.at[pl.ds(k * SUB, SUB)], sem)
               for k in range(CB // SUB)]
        for cp in cps:
            cp.wait()

        def row_body(k, carry2):
            row0 = k * LANES
            rows16 = row0 + iota16
            info16 = plsc.load_gather(info_all,
                                      [iota_l + (cbase + k * (LANES * L))])
            off_k = (row0 // 128) * 1024 + (row0 % 128)
            ss = [jnp.zeros((LANES,), jnp.float32) for _ in range(4)]
            qq = [jnp.zeros((LANES,), jnp.float32) for _ in range(4)]
            for j in range(E):
                colj = jnp.full((LANES,), j, jnp.int32)
                v = plsc.load_gather(rows_v, [rows16, colj])
                ss[j % 4] = ss[j % 4] + v
                qq[j % 4] = qq[j % 4] + v * v
            svec = (ss[0] + ss[1]) + (ss[2] + ss[3])
            qvec = (qq[0] + qq[1]) + (qq[2] + qq[3])
            mean = svec * (1.0 / E)
            var_t = qvec * (1.0 / E) - mean * mean
            vy = jnp.maximum(info16 * info16 * var_t + LN_EPS, 1e-30)
            a_vec = info16 * _rsqrt(vy)
            b_vec = -mean * a_vec
            for u in range(LANES):
                r = row0 + u
                abc = jnp.full((LANES,), a_vec[u], jnp.float32)
                bbc = jnp.full((LANES,), b_vec[u], jnp.float32)
                xs = [rows_v[r, pl.ds(p * LANES, LANES)] for p in range(EV)]
                for p in range(EV):
                    y = (xs[p] * abc + bbc) * gammas[p] + betas[p]
                    plsc.store_scatter(stg_v, [svecs[p] + (off_k + u)], y)
            return carry2

        lax.fori_loop(0, NBLK, row_body, 0)

        btile0 = wid * (BW // 128) + sb * (CB // 128)
        for tr in range(E // 8):
            pltpu.sync_copy(
                stg_v.at[pl.ds(tr * 2048, 2048)],
                out_hbm.at[l, pl.ds(tr * (1024 * B // 128) + btile0 * 1024,
                                    2048)])
        return carry

    lax.fori_loop(0, 2 * L, chunk_body, 0)


def kernel(holder, info, table, ln_gamma, ln_beta):
    b, l = holder.shape
    v, e = table.shape
    n = b * l
    assert (b, l, e) == (B, L, E)
    idx = holder.reshape(n).astype(jnp.int32)
    infof = info.reshape(n)

    mesh = plsc.VectorSubcoreMesh(core_axis_name="c", subcore_axis_name="s",
                                  num_cores=NC, num_subcores=NS)
    run = pl.kernel(
        _body,
        out_type=jax.ShapeDtypeStruct((L, (E // 8) * (B // 128) * 8 * 128),
                                      jnp.float32),
        mesh=mesh,
        scratch_types=[
            pltpu.VMEM((BW * L,), jnp.int32),
            pltpu.VMEM((BW * L,), jnp.float32),
            pltpu.VMEM((CB,), jnp.int32),
            pltpu.VMEM((CB, E), jnp.float32),
            pltpu.VMEM((2, E), jnp.float32),
            pltpu.VMEM(((E // 8) * (CB // 128) * 8 * 128,), jnp.float32),
            pltpu.VMEM((2 * LANES, LANES), jnp.float32),
            pltpu.VMEM((2 * LANES,), jnp.float32),
            pltpu.SemaphoreType.DMA,
        ],
        compiler_params=pltpu.CompilerParams(needs_layout_passes=False,
                                             use_tc_tiling_on_sc=False),
    )
    out2 = run(idx, infof, table, ln_gamma, ln_beta)
    out5 = out2.reshape(L, E // 8, B // 128, 8, 128)
    return out5.transpose(2, 4, 0, 1, 3).reshape(B, L, E)


# speedup vs baseline: 1.1357x; 1.1357x over previous
import jax
import jax.numpy as jnp
from jax import lax
from jax.experimental import pallas as pl
from jax.experimental.pallas import tpu as pltpu
from jax.experimental.pallas import tpu_sc as plsc

NC = 2
NS = 16
NW = NC * NS
LANES = 16

B = 16384
L = 50
E = 64
BW = B // NW
CB = 512
NBLK = CB // LANES
SUB = 128
LN_EPS = 1e-5


def _rsqrt(x):
    i = plsc.bitcast(x, jnp.int32)
    y = plsc.bitcast(jnp.int32(0x5F3759DF) - (i >> 1), jnp.float32)
    for _ in range(3):
        y = y * (1.5 - 0.5 * x * y * y)
    return y


def _body(idx_hbm, info_hbm, table_hbm, gamma_hbm, beta_hbm, out_hbm,
          idx_all, info_all, idx_buf, rows_v, a_v, b_v, gb_v, gbb_v, stg_v,
          sem):
    wid = lax.axis_index("s") * NC + lax.axis_index("c")
    nw_base = pl.multiple_of(wid * (BW * L), BW * L)

    pltpu.sync_copy(idx_hbm.at[pl.ds(nw_base, BW * L)], idx_all)
    pltpu.sync_copy(info_hbm.at[pl.ds(nw_base, BW * L)], info_all)
    pltpu.sync_copy(gamma_hbm, gb_v.at[0])
    pltpu.sync_copy(beta_hbm, gb_v.at[1])

    iota16 = lax.iota(jnp.int32, LANES)
    iota_l = iota16 * L

    def bcast_body(e, carry):
        ebc = jnp.full((LANES,), e, jnp.int32)
        gbb_v[e, pl.ds(0, LANES)] = plsc.load_gather(gb_v, [jnp.zeros((LANES,), jnp.int32), ebc])
        gbb_v[E + e, pl.ds(0, LANES)] = plsc.load_gather(gb_v, [jnp.ones((LANES,), jnp.int32), ebc])
        return carry

    lax.fori_loop(0, E, bcast_body, 0)

    def chunk_body(c, carry):
        l = c
        sb = 0
        cbase = l

        def idxb_body(k, carry2):
            pvec = iota_l + (cbase + k * (LANES * L))
            idx_buf[pl.ds(k * LANES, LANES)] = plsc.load_gather(idx_all, [pvec])
            return carry2

        lax.fori_loop(0, NBLK, idxb_body, 0)

        cps = [pltpu.async_copy(table_hbm.at[idx_buf.at[pl.ds(k * SUB, SUB)]],
                                rows_v.at[pl.ds(k * SUB, SUB)], sem)
               for k in range(CB // SUB)]
        for cp in cps:
            cp.wait()

        def stats_body(k, carry2):
            row0 = k * LANES
            rows16 = row0 + iota16
            ss = [jnp.zeros((LANES,), jnp.float32) for _ in range(4)]
            qq = [jnp.zeros((LANES,), jnp.float32) for _ in range(4)]
            for j in range(E):
                colj = jnp.full((LANES,), j, jnp.int32)
                v = plsc.load_gather(rows_v, [rows16, colj])
                ss[j % 4] = ss[j % 4] + v
                qq[j % 4] = qq[j % 4] + v * v
            s = (ss[0] + ss[1]) + (ss[2] + ss[3])
            s2 = (qq[0] + qq[1]) + (qq[2] + qq[3])
            mean = s * (1.0 / E)
            var_t = s2 * (1.0 / E) - mean * mean
            pvec = iota_l + (cbase + k * (LANES * L))
            infov = plsc.load_gather(info_all, [pvec])
            vy = jnp.maximum(infov * infov * var_t + LN_EPS, 1e-30)
            a = infov * _rsqrt(vy)
            a_v[pl.ds(row0, LANES)] = a
            b_v[pl.ds(row0, LANES)] = -mean * a
            return carry2

        lax.fori_loop(0, NBLK, stats_body, 0)

        def apply_body(k, carry2):
            row0 = k * LANES
            rows16 = row0 + iota16
            av = a_v[pl.ds(row0, LANES)]
            bv = b_v[pl.ds(row0, LANES)]
            tc = k // 8
            boff = (k % 8) * LANES
            for e0 in range(0, E, 8):
                es = list(range(e0, e0 + 8))
                xs = [plsc.load_gather(rows_v,
                                       [rows16, jnp.full((LANES,), e, jnp.int32)])
                      for e in es]
                zs = [x * av + bv for x in xs]
                for e, z in zip(es, zs):
                    stg_v[e // 8, tc, e % 8, pl.ds(boff, LANES)] = z
            return carry2

        lax.fori_loop(0, NBLK, apply_body, 0)

        def gb_body(g, carry2):
            tr = g // 8
            e8 = g % 8
            ge = gbb_v[g, pl.ds(0, LANES)]
            be = gbb_v[E + g, pl.ds(0, LANES)]
            for tc2 in range(CB // 128):
                zs = [stg_v[tr, tc2, e8, pl.ds(s * LANES, LANES)]
                      for s in range(8)]
                ys = [z * ge + be for z in zs]
                for s, y in enumerate(ys):
                    stg_v[tr, tc2, e8, pl.ds(s * LANES, LANES)] = y
            return carry2

        lax.fori_loop(0, E, gb_body, 0)

        btile0 = wid * (BW // 128) + sb * (CB // 128)
        pltpu.sync_copy(stg_v, out_hbm.at[l, :, pl.ds(btile0, CB // 128)])
        return carry

    lax.fori_loop(0, L, chunk_body, 0)


def kernel(holder, info, table, ln_gamma, ln_beta):
    b, l = holder.shape
    v, e = table.shape
    n = b * l
    assert (b, l, e) == (B, L, E)
    idx = holder.reshape(n).astype(jnp.int32)
    infof = info.reshape(n)

    mesh = plsc.VectorSubcoreMesh(core_axis_name="c", subcore_axis_name="s",
                                  num_cores=NC, num_subcores=NS)
    run = pl.kernel(
        _body,
        out_type=jax.ShapeDtypeStruct((L, E // 8, B // 128, 8, 128), jnp.float32),
        mesh=mesh,
        scratch_types=[
            pltpu.VMEM((BW * L,), jnp.int32),
            pltpu.VMEM((BW * L,), jnp.float32),
            pltpu.VMEM((CB,), jnp.int32),
            pltpu.VMEM((CB, E), jnp.float32),
            pltpu.VMEM((CB,), jnp.float32),
            pltpu.VMEM((CB,), jnp.float32),
            pltpu.VMEM((2, E), jnp.float32),
            pltpu.VMEM((2 * E, LANES), jnp.float32),
            pltpu.VMEM((E // 8, CB // 128, 8, 128), jnp.float32),
            pltpu.SemaphoreType.DMA,
        ],
        compiler_params=pltpu.CompilerParams(needs_layout_passes=False,
                                             use_tc_tiling_on_sc=False),
    )
    out5 = run(idx, infof, table, ln_gamma, ln_beta)
    return out5.transpose(2, 4, 0, 1, 3).reshape(B, L, E)


# speedup vs baseline: 1.1753x; 1.0349x over previous
import jax
import jax.numpy as jnp
from jax import lax
from jax.experimental import pallas as pl
from jax.experimental.pallas import tpu as pltpu
from jax.experimental.pallas import tpu_sc as plsc

NC = 2
NS = 16
NW = NC * NS
LANES = 16

B = 16384
L = 50
E = 64
BW = B // NW
CB = 256
NCHUNK = (BW // CB) * L
NBLK = CB // LANES
SUB = 128
LN_EPS = 1e-5


def _rsqrt(x):
    i = plsc.bitcast(x, jnp.int32)
    y = plsc.bitcast(jnp.int32(0x5F3759DF) - (i >> 1), jnp.float32)
    for _ in range(3):
        y = y * (1.5 - 0.5 * x * y * y)
    return y


def _body(idx_hbm, info_hbm, table_hbm, gamma_hbm, beta_hbm, out_hbm,
          idx_all, info_all, idx0, idx1, rows0, rows1, a_v, b_v, gb_v, gbb_v,
          stg_v, sem0, sem1):
    wid = lax.axis_index("s") * NC + lax.axis_index("c")
    nw_base = pl.multiple_of(wid * (BW * L), BW * L)

    pltpu.sync_copy(idx_hbm.at[pl.ds(nw_base, BW * L)], idx_all)
    pltpu.sync_copy(info_hbm.at[pl.ds(nw_base, BW * L)], info_all)
    pltpu.sync_copy(gamma_hbm, gb_v.at[0])
    pltpu.sync_copy(beta_hbm, gb_v.at[1])

    iota16 = lax.iota(jnp.int32, LANES)
    iota_l = iota16 * L

    def bcast_body(e, carry):
        ebc = jnp.full((LANES,), e, jnp.int32)
        gbb_v[e, pl.ds(0, LANES)] = plsc.load_gather(
            gb_v, [jnp.zeros((LANES,), jnp.int32), ebc])
        gbb_v[E + e, pl.ds(0, LANES)] = plsc.load_gather(
            gb_v, [jnp.ones((LANES,), jnp.int32), ebc])
        return carry

    lax.fori_loop(0, E, bcast_body, 0)

    def cbase_of(c):
        return (c % 2) * (CB * L) + c // 2

    def build_idx(c, idx_ref):
        cbase = cbase_of(c)

        def idxb_body(k, carry2):
            pvec = iota_l + (cbase + k * (LANES * L))
            idx_ref[pl.ds(k * LANES, LANES)] = plsc.load_gather(idx_all, [pvec])
            return carry2

        lax.fori_loop(0, NBLK, idxb_body, 0)

    def fire(idx_ref, rows_ref, sem):
        for k in range(CB // SUB):
            pltpu.async_copy(table_hbm.at[idx_ref.at[pl.ds(k * SUB, SUB)]],
                             rows_ref.at[pl.ds(k * SUB, SUB)], sem)

    def drain(idx_ref, rows_ref, sem):
        for k in range(CB // SUB):
            pltpu.make_async_copy(
                table_hbm.at[idx_ref.at[pl.ds(k * SUB, SUB)]],
                rows_ref.at[pl.ds(k * SUB, SUB)], sem).wait()

    def process(c, rows_ref):
        l = c // 2
        sb = c % 2
        cbase = cbase_of(c)

        def stats_body(k, carry2):
            row0 = k * LANES
            rows16 = row0 + iota16
            ss = [jnp.zeros((LANES,), jnp.float32) for _ in range(4)]
            qq = [jnp.zeros((LANES,), jnp.float32) for _ in range(4)]
            for j in range(E):
                colj = jnp.full((LANES,), j, jnp.int32)
                v = plsc.load_gather(rows_ref, [rows16, colj])
                ss[j % 4] = ss[j % 4] + v
                qq[j % 4] = qq[j % 4] + v * v
            s = (ss[0] + ss[1]) + (ss[2] + ss[3])
            s2 = (qq[0] + qq[1]) + (qq[2] + qq[3])
            mean = s * (1.0 / E)
            var_t = s2 * (1.0 / E) - mean * mean
            pvec = iota_l + (cbase + k * (LANES * L))
            infov = plsc.load_gather(info_all, [pvec])
            vy = jnp.maximum(infov * infov * var_t + LN_EPS, 1e-30)
            a = infov * _rsqrt(vy)
            a_v[pl.ds(row0, LANES)] = a
            b_v[pl.ds(row0, LANES)] = -mean * a
            return carry2

        lax.fori_loop(0, NBLK, stats_body, 0)

        def apply_body(k, carry2):
            row0 = k * LANES
            rows16 = row0 + iota16
            av = a_v[pl.ds(row0, LANES)]
            bv = b_v[pl.ds(row0, LANES)]
            tc = k // 8
            boff = (k % 8) * LANES
            for e0 in range(0, E, 8):
                es = list(range(e0, e0 + 8))
                xs = [plsc.load_gather(
                          rows_ref,
                          [rows16, jnp.full((LANES,), e, jnp.int32)])
                      for e in es]
                zs = [x * av + bv for x in xs]
                for e, z in zip(es, zs):
                    stg_v[e // 8, tc, e % 8, pl.ds(boff, LANES)] = z
            return carry2

        lax.fori_loop(0, NBLK, apply_body, 0)

        def gb_body(g, carry2):
            tr = g // 8
            e8 = g % 8
            ge = gbb_v[g, pl.ds(0, LANES)]
            be = gbb_v[E + g, pl.ds(0, LANES)]
            for tc2 in range(CB // 128):
                zs = [stg_v[tr, tc2, e8, pl.ds(s * LANES, LANES)]
                      for s in range(8)]
                ys = [z * ge + be for z in zs]
                for s, y in enumerate(ys):
                    stg_v[tr, tc2, e8, pl.ds(s * LANES, LANES)] = y
            return carry2

        lax.fori_loop(0, E, gb_body, 0)

        btile0 = wid * (BW // 128) + sb * (CB // 128)
        pltpu.sync_copy(stg_v, out_hbm.at[l, :, pl.ds(btile0, CB // 128)])

    build_idx(0, idx0)
    fire(idx0, rows0, sem0)

    def pair_body(t, carry):
        c0 = 2 * t
        c1 = 2 * t + 1
        build_idx(c1, idx1)
        fire(idx1, rows1, sem1)
        drain(idx0, rows0, sem0)
        process(c0, rows0)
        c2 = jnp.minimum(c0 + 2, NCHUNK - 1)
        build_idx(c2, idx0)
        fire(idx0, rows0, sem0)
        drain(idx1, rows1, sem1)
        process(c1, rows1)
        return carry

    lax.fori_loop(0, NCHUNK // 2, pair_body, 0)
    drain(idx0, rows0, sem0)


def kernel(holder, info, table, ln_gamma, ln_beta):
    b, l = holder.shape
    v, e = table.shape
    n = b * l
    assert (b, l, e) == (B, L, E)
    idx = holder.reshape(n).astype(jnp.int32)
    infof = info.reshape(n)

    mesh = plsc.VectorSubcoreMesh(core_axis_name="c", subcore_axis_name="s",
                                  num_cores=NC, num_subcores=NS)
    run = pl.kernel(
        _body,
        out_type=jax.ShapeDtypeStruct((L, E // 8, B // 128, 8, 128),
                                      jnp.float32),
        mesh=mesh,
        scratch_types=[
            pltpu.VMEM((BW * L,), jnp.int32),
            pltpu.VMEM((BW * L,), jnp.float32),
            pltpu.VMEM((CB,), jnp.int32),
            pltpu.VMEM((CB,), jnp.int32),
            pltpu.VMEM((CB, E), jnp.float32),
            pltpu.VMEM((CB, E), jnp.float32),
            pltpu.VMEM((CB,), jnp.float32),
            pltpu.VMEM((CB,), jnp.float32),
            pltpu.VMEM((2, E), jnp.float32),
            pltpu.VMEM((2 * E, LANES), jnp.float32),
            pltpu.VMEM((E // 8, CB // 128, 8, 128), jnp.float32),
            pltpu.SemaphoreType.DMA,
            pltpu.SemaphoreType.DMA,
        ],
        compiler_params=pltpu.CompilerParams(needs_layout_passes=False,
                                             use_tc_tiling_on_sc=False),
    )
    out5 = run(idx, infof, table, ln_gamma, ln_beta)
    return out5.transpose(2, 4, 0, 1, 3).reshape(B, L, E)


# speedup vs baseline: 1.2114x; 1.0307x over previous
import jax
import jax.numpy as jnp
from jax import lax
from jax.experimental import pallas as pl
from jax.experimental.pallas import tpu as pltpu
from jax.experimental.pallas import tpu_sc as plsc

NC = 2
NS = 16
NW = NC * NS
LANES = 16

B = 16384
L = 50
E = 64
BW = B // NW
CB = 256
NCHUNK = (BW // CB) * L
NBLK = CB // LANES
SUB = 128
LN_EPS = 1e-5


def _rsqrt(x):
    i = plsc.bitcast(x, jnp.int32)
    y = plsc.bitcast(jnp.int32(0x5F3759DF) - (i >> 1), jnp.float32)
    for _ in range(3):
        y = y * (1.5 - 0.5 * x * y * y)
    return y


def _body(idx_hbm, info_hbm, table_hbm, gamma_hbm, beta_hbm, out_hbm,
          idx_all, info_all, idx0, idx1, rows0, rows1, a_v, b_v, gb_v, gbb_v,
          stg0, stg1, sem0, sem1, semo0, semo1):
    wid = lax.axis_index("s") * NC + lax.axis_index("c")
    nw_base = pl.multiple_of(wid * (BW * L), BW * L)

    pltpu.sync_copy(idx_hbm.at[pl.ds(nw_base, BW * L)], idx_all)
    pltpu.sync_copy(info_hbm.at[pl.ds(nw_base, BW * L)], info_all)
    pltpu.sync_copy(gamma_hbm, gb_v.at[0])
    pltpu.sync_copy(beta_hbm, gb_v.at[1])

    iota16 = lax.iota(jnp.int32, LANES)
    iota_l = iota16 * L

    def bcast_body(e, carry):
        ebc = jnp.full((LANES,), e, jnp.int32)
        gbb_v[e, pl.ds(0, LANES)] = plsc.load_gather(
            gb_v, [jnp.zeros((LANES,), jnp.int32), ebc])
        gbb_v[E + e, pl.ds(0, LANES)] = plsc.load_gather(
            gb_v, [jnp.ones((LANES,), jnp.int32), ebc])
        return carry

    lax.fori_loop(0, E, bcast_body, 0)

    def cbase_of(c):
        return (c % 2) * (CB * L) + c // 2

    def build_idx(c, idx_ref):
        cbase = cbase_of(c)

        def idxb_body(k, carry2):
            pvec = iota_l + (cbase + k * (LANES * L))
            idx_ref[pl.ds(k * LANES, LANES)] = plsc.load_gather(idx_all, [pvec])
            return carry2

        lax.fori_loop(0, NBLK, idxb_body, 0)

    def fire(idx_ref, rows_ref, sem):
        for k in range(CB // SUB):
            pltpu.async_copy(table_hbm.at[idx_ref.at[pl.ds(k * SUB, SUB)]],
                             rows_ref.at[pl.ds(k * SUB, SUB)], sem)

    def drain(idx_ref, rows_ref, sem):
        for k in range(CB // SUB):
            pltpu.make_async_copy(
                table_hbm.at[idx_ref.at[pl.ds(k * SUB, SUB)]],
                rows_ref.at[pl.ds(k * SUB, SUB)], sem).wait()

    def process(c, rows_ref, stg_v, semo, first):
        l = c // 2
        sb = c % 2
        cbase = cbase_of(c)
        btile0 = wid * (BW // 128) + sb * (CB // 128)
        out_dst = out_hbm.at[l, :, pl.ds(btile0, CB // 128)]

        def stats_body(k, carry2):
            row0 = k * LANES
            rows16 = row0 + iota16
            ss = [jnp.zeros((LANES,), jnp.float32) for _ in range(4)]
            qq = [jnp.zeros((LANES,), jnp.float32) for _ in range(4)]
            for j in range(E):
                colj = jnp.full((LANES,), j, jnp.int32)
                v = plsc.load_gather(rows_ref, [rows16, colj])
                ss[j % 4] = ss[j % 4] + v
                qq[j % 4] = qq[j % 4] + v * v
            s = (ss[0] + ss[1]) + (ss[2] + ss[3])
            s2 = (qq[0] + qq[1]) + (qq[2] + qq[3])
            mean = s * (1.0 / E)
            var_t = s2 * (1.0 / E) - mean * mean
            pvec = iota_l + (cbase + k * (LANES * L))
            infov = plsc.load_gather(info_all, [pvec])
            vy = jnp.maximum(infov * infov * var_t + LN_EPS, 1e-30)
            a = infov * _rsqrt(vy)
            a_v[pl.ds(row0, LANES)] = a
            b_v[pl.ds(row0, LANES)] = -mean * a
            return carry2

        lax.fori_loop(0, NBLK, stats_body, 0)

        @pl.when(jnp.logical_not(first))
        def _():
            pltpu.make_async_copy(stg_v, out_dst, semo).wait()

        def apply_body(k, carry2):
            row0 = k * LANES
            rows16 = row0 + iota16
            av = a_v[pl.ds(row0, LANES)]
            bv = b_v[pl.ds(row0, LANES)]
            tc = k // 8
            boff = (k % 8) * LANES
            for e0 in range(0, E, 8):
                es = list(range(e0, e0 + 8))
                xs = [plsc.load_gather(
                          rows_ref,
                          [rows16, jnp.full((LANES,), e, jnp.int32)])
                      for e in es]
                zs = [x * av + bv for x in xs]
                for e, z in zip(es, zs):
                    stg_v[e // 8, tc, e % 8, pl.ds(boff, LANES)] = z
            return carry2

        lax.fori_loop(0, NBLK, apply_body, 0)

        def gb_body(g, carry2):
            tr = g // 8
            e8 = g % 8
            ge = gbb_v[g, pl.ds(0, LANES)]
            be = gbb_v[E + g, pl.ds(0, LANES)]
            for tc2 in range(CB // 128):
                zs = [stg_v[tr, tc2, e8, pl.ds(s * LANES, LANES)]
                      for s in range(8)]
                ys = [z * ge + be for z in zs]
                for s, y in enumerate(ys):
                    stg_v[tr, tc2, e8, pl.ds(s * LANES, LANES)] = y
            return carry2

        lax.fori_loop(0, E, gb_body, 0)

        pltpu.async_copy(stg_v, out_dst, semo)

    build_idx(0, idx0)
    fire(idx0, rows0, sem0)

    def pair_body(t, carry):
        c0 = 2 * t
        c1 = 2 * t + 1
        first = t == 0
        build_idx(c1, idx1)
        fire(idx1, rows1, sem1)
        drain(idx0, rows0, sem0)
        process(c0, rows0, stg0, semo0, first)
        c2 = jnp.minimum(c0 + 2, NCHUNK - 1)
        build_idx(c2, idx0)
        fire(idx0, rows0, sem0)
        drain(idx1, rows1, sem1)
        process(c1, rows1, stg1, semo1, first)
        return carry

    lax.fori_loop(0, NCHUNK // 2, pair_body, 0)
    drain(idx0, rows0, sem0)
    pltpu.make_async_copy(stg0, out_hbm.at[0, :, pl.ds(0, CB // 128)], semo0).wait()
    pltpu.make_async_copy(stg1, out_hbm.at[0, :, pl.ds(0, CB // 128)], semo1).wait()


def kernel(holder, info, table, ln_gamma, ln_beta):
    b, l = holder.shape
    v, e = table.shape
    n = b * l
    assert (b, l, e) == (B, L, E)
    idx = holder.reshape(n).astype(jnp.int32)
    infof = info.reshape(n)

    mesh = plsc.VectorSubcoreMesh(core_axis_name="c", subcore_axis_name="s",
                                  num_cores=NC, num_subcores=NS)
    run = pl.kernel(
        _body,
        out_type=jax.ShapeDtypeStruct((L, E // 8, B // 128, 8, 128),
                                      jnp.float32),
        mesh=mesh,
        scratch_types=[
            pltpu.VMEM((BW * L,), jnp.int32),
            pltpu.VMEM((BW * L,), jnp.float32),
            pltpu.VMEM((CB,), jnp.int32),
            pltpu.VMEM((CB,), jnp.int32),
            pltpu.VMEM((CB, E), jnp.float32),
            pltpu.VMEM((CB, E), jnp.float32),
            pltpu.VMEM((CB,), jnp.float32),
            pltpu.VMEM((CB,), jnp.float32),
            pltpu.VMEM((2, E), jnp.float32),
            pltpu.VMEM((2 * E, LANES), jnp.float32),
            pltpu.VMEM((E // 8, CB // 128, 8, 128), jnp.float32),
            pltpu.VMEM((E // 8, CB // 128, 8, 128), jnp.float32),
            pltpu.SemaphoreType.DMA,
            pltpu.SemaphoreType.DMA,
            pltpu.SemaphoreType.DMA,
            pltpu.SemaphoreType.DMA,
        ],
        compiler_params=pltpu.CompilerParams(needs_layout_passes=False,
                                             use_tc_tiling_on_sc=False),
    )
    out5 = run(idx, infof, table, ln_gamma, ln_beta)
    return out5.transpose(2, 4, 0, 1, 3).reshape(B, L, E)
